# baseline (speedup 1.0000x reference)
import numpy as np
import jax
import jax.numpy as jnp
from jax.experimental import pallas as pl
from jax.experimental.pallas import tpu as pltpu

_EPS = 1e-06
_N = 128
_BB = 8

_II, _JJ = np.tril_indices(_N, k=-1)

_PP = np.arange(_N)[:, None]
_LL = np.arange(_N)[None, :]
_P = (((_PP >> 5) == (_LL & 3)) & ((_PP & 31) == (_LL >> 2))
      ).astype(np.float32)


def _feat_kernel(pt_ref, eta_ref, phi_ref, en_ref, msk_ref, p_ref, out_ref):
    pt = pt_ref[...]
    eta = eta_ref[...]
    phi = phi_ref[...]
    en = en_ref[...]
    keep = 1.0 - msk_ref[...]
    perm = p_ref[...]

    t = jnp.exp(eta)
    pz = pt * (0.5 * (t - 1.0 / t))
    e_plus = jnp.clip(en + pz, _EPS, None)
    e_minus = jnp.clip(en - pz, _EPS, None)
    rap = 0.5 * jnp.log(jnp.clip(e_plus / e_minus, _EPS, None))
    px = pt * jnp.cos(phi)
    py = pt * jnp.sin(phi)

    phi_t = phi.T
    rap_t = rap.T
    pt_t = pt.T
    px_t = px.T
    py_t = py.T
    pz_t = pz.T
    en_t = en.T
    keep_t = keep.T

    n = _N
    row_ids = jax.lax.broadcasted_iota(jnp.int32, (n, n), 0)
    col_ids = jax.lax.broadcasted_iota(jnp.int32, (n, n), 1)
    offdiag = (row_ids != col_ids).astype(jnp.float32)

    for r in range(_BB):
        def rowmat(v):
            return jnp.broadcast_to(v[r].reshape(1, n), (n, n))

        def colmat(vt):
            return jnp.broadcast_to(vt[:, r].reshape(n, 1), (n, n))

        phi_j = rowmat(phi)
        rap_j = rowmat(rap)
        pt_j = rowmat(pt)
        px_j = rowmat(px)
        py_j = rowmat(py)
        pz_j = rowmat(pz)
        e_j = rowmat(en)
        keep_j = rowmat(keep)

        phi_i = colmat(phi_t)
        rap_i = colmat(rap_t)
        pt_i = colmat(pt_t)
        px_i = colmat(px_t)
        py_i = colmat(py_t)
        pz_i = colmat(pz_t)
        e_i = colmat(en_t)
        keep_i = colmat(keep_t)

        dphi = jnp.mod(phi_i - phi_j + jnp.pi, 2.0 * jnp.pi) - jnp.pi
        drap = rap_i - rap_j
        dr = jnp.sqrt(drap * drap + dphi * dphi)
        dr = jnp.log(1.0 + jnp.clip(dr, _EPS, None))
        minpt = jnp.minimum(pt_i, pt_j)
        kt = jnp.log(1.0 + jnp.clip(minpt * dr, _EPS, None))
        z = jnp.log(1.0 + jnp.clip(minpt / (pt_i + pt_j + _EPS), _EPS, None))
        se = e_i + e_j
        spx = px_i + px_j
        spy = py_i + py_j
        spz = pz_i + pz_j
        m2 = jnp.log(1.0 + jnp.clip(
            se * se - spx * spx - spy * spy - spz * spz, _EPS, None))

        scale = offdiag * keep_i * keep_j
        feats = [dr * scale, kt * scale, z * scale, m2 * scale]
        f_hi = [f.astype(jnp.bfloat16) for f in feats]
        f_lo = [(f - h.astype(jnp.float32)).astype(jnp.bfloat16)
                for f, h in zip(feats, f_hi)]
        for s in range(4):
            sl = slice(32 * s, 32 * (s + 1))
            xs_hi = jnp.concatenate([h[:, sl] for h in f_hi], axis=1)
            xs_lo = jnp.concatenate([l[:, sl] for l in f_lo], axis=1)
            out_ref[r, :, 128 * s:128 * (s + 1)] = (
                jnp.dot(xs_hi, perm, preferred_element_type=jnp.float32)
                + jnp.dot(xs_lo, perm, preferred_element_type=jnp.float32))


def kernel(pt, eta, phi, energy, mask):
    b, n = pt.shape
    maskf = mask.astype(jnp.float32)
    permb = jnp.asarray(_P, dtype=jnp.bfloat16)
    bspec_in = pl.BlockSpec((_BB, n), lambda g: (g, 0))
    bspec_p = pl.BlockSpec((n, n), lambda g: (0, 0))
    out = pl.pallas_call(
        _feat_kernel,
        grid=(b // _BB,),
        in_specs=[bspec_in] * 5 + [bspec_p],
        out_specs=pl.BlockSpec((_BB, n, 4 * n), lambda g: (g, 0, 0)),
        out_shape=jax.ShapeDtypeStruct((b, n, 4 * n), jnp.float32),
        compiler_params=pltpu.CompilerParams(
            dimension_semantics=("arbitrary",)),
    )(pt, eta, phi, energy, maskf, permb)
    features = out.reshape(b, n, n, 4)
    pair_mask = mask[:, _II] | mask[:, _JJ]
    return features, pair_mask


# speedup vs baseline: 1.5246x; 1.5246x over previous
import numpy as np
import jax
import jax.numpy as jnp
from jax.experimental import pallas as pl
from jax.experimental.pallas import tpu as pltpu

_EPS = 1e-06
_N = 128
_BB = 8

_II, _JJ = np.tril_indices(_N, k=-1)


def _feat_kernel(pt_ref, eta_ref, phi_ref, en_ref, msk_ref,
                 dr_ref, kt_ref, z_ref, m2_ref):
    pt = pt_ref[...]
    eta = eta_ref[...]
    phi = phi_ref[...]
    en = en_ref[...]
    keep = 1.0 - msk_ref[...]

    t = jnp.exp(eta)
    pz = pt * (0.5 * (t - 1.0 / t))
    e_plus = jnp.clip(en + pz, _EPS, None)
    e_minus = jnp.clip(en - pz, _EPS, None)
    rap = 0.5 * jnp.log(jnp.clip(e_plus / e_minus, _EPS, None))
    px = pt * jnp.cos(phi)
    py = pt * jnp.sin(phi)

    phi_t = phi.T
    rap_t = rap.T
    pt_t = pt.T
    px_t = px.T
    py_t = py.T
    pz_t = pz.T
    en_t = en.T
    keep_t = keep.T

    n = _N
    row_ids = jax.lax.broadcasted_iota(jnp.int32, (n, n), 0)
    col_ids = jax.lax.broadcasted_iota(jnp.int32, (n, n), 1)
    offdiag = (row_ids != col_ids).astype(jnp.float32)
    lowm = row_ids > col_ids

    def rowmat(v, r):
        return jnp.broadcast_to(v[r].reshape(1, n), (n, n))

    def colmat(vt, r):
        return jnp.broadcast_to(vt[:, r].reshape(n, 1), (n, n))

    for m in range(_BB // 2):
        a = 2 * m
        b = 2 * m + 1

        def mixrow(v):
            return jnp.where(lowm, rowmat(v, a), rowmat(v, b))

        def mixcol(vt):
            return jnp.where(lowm, colmat(vt, a), colmat(vt, b))

        phi_j = mixrow(phi)
        rap_j = mixrow(rap)
        pt_j = mixrow(pt)
        px_j = mixrow(px)
        py_j = mixrow(py)
        pz_j = mixrow(pz)
        e_j = mixrow(en)

        phi_i = mixcol(phi_t)
        rap_i = mixcol(rap_t)
        pt_i = mixcol(pt_t)
        px_i = mixcol(px_t)
        py_i = mixcol(py_t)
        pz_i = mixcol(pz_t)
        e_i = mixcol(en_t)

        dphi = jnp.mod(phi_i - phi_j + jnp.pi, 2.0 * jnp.pi) - jnp.pi
        drap = rap_i - rap_j
        dr = jnp.sqrt(drap * drap + dphi * dphi)
        dr = jnp.log(1.0 + jnp.clip(dr, _EPS, None))
        minpt = jnp.minimum(pt_i, pt_j)
        kt = jnp.log(1.0 + jnp.clip(minpt * dr, _EPS, None))
        z = jnp.log(1.0 + jnp.clip(minpt / (pt_i + pt_j + _EPS), _EPS, None))
        se = e_i + e_j
        spx = px_i + px_j
        spy = py_i + py_j
        spz = pz_i + pz_j
        m2 = jnp.log(1.0 + jnp.clip(
            se * se - spx * spx - spy * spy - spz * spz, _EPS, None))

        scale_a = offdiag * colmat(keep_t, a) * rowmat(keep, a)
        scale_b = offdiag * colmat(keep_t, b) * rowmat(keep, b)

        for f, ref in ((dr, dr_ref), (kt, kt_ref), (z, z_ref), (m2, m2_ref)):
            ft = f.T
            ref[a] = jnp.where(lowm, f, ft) * scale_a
            ref[b] = jnp.where(lowm, ft, f) * scale_b


def kernel(pt, eta, phi, energy, mask):
    b, n = pt.shape
    maskf = mask.astype(jnp.float32)
    bspec_in = pl.BlockSpec((_BB, n), lambda g: (g, 0))
    bspec_out = pl.BlockSpec((_BB, n, n), lambda g: (g, 0, 0))
    shp = jax.ShapeDtypeStruct((b, n, n), jnp.float32)
    dr, kt, z, m2 = pl.pallas_call(
        _feat_kernel,
        grid=(b // _BB,),
        in_specs=[bspec_in] * 5,
        out_specs=[bspec_out] * 4,
        out_shape=[shp] * 4,
    )(pt, eta, phi, energy, maskf)
    features = jnp.stack([dr, kt, z, m2], axis=-1)
    pair_mask = mask[:, _II] | mask[:, _JJ]
    return features, pair_mask


# speedup vs baseline: 1.7039x; 1.1176x over previous
import numpy as np
import jax
import jax.numpy as jnp
from jax.experimental import pallas as pl
from jax.experimental.pallas import tpu as pltpu

_EPS = 1e-06
_N = 128
_BB = 8

_II, _JJ = np.tril_indices(_N, k=-1)


def _feat_kernel(pt_ref, eta_ref, phi_ref, en_ref, msk_ref,
                 dr_ref, kt_ref, z_ref, m2_ref):
    pt = pt_ref[...]
    eta = eta_ref[...]
    phi = phi_ref[...]
    en = en_ref[...]
    keep = 1.0 - msk_ref[...]

    t = jnp.exp(eta)
    pz = pt * (0.5 * (t - 1.0 / t))
    e_plus = jnp.clip(en + pz, _EPS, None)
    e_minus = jnp.clip(en - pz, _EPS, None)
    rap = 0.5 * jnp.log(jnp.clip(e_plus / e_minus, _EPS, None))
    px = pt * jnp.cos(phi)
    py = pt * jnp.sin(phi)

    phi_t = phi.T
    rap_t = rap.T
    pt_t = pt.T
    px_t = px.T
    py_t = py.T
    pz_t = pz.T
    en_t = en.T
    keep_t = keep.T

    n = _N
    row_ids = jax.lax.broadcasted_iota(jnp.int32, (n, n), 0)
    col_ids = jax.lax.broadcasted_iota(jnp.int32, (n, n), 1)
    offdiag = (row_ids != col_ids).astype(jnp.float32)

    for r in range(_BB):
        def rowmat(v):
            return jnp.broadcast_to(v[r].reshape(1, n), (n, n))

        def colmat(vt):
            return jnp.broadcast_to(vt[:, r].reshape(n, 1), (n, n))

        phi_j = rowmat(phi)
        rap_j = rowmat(rap)
        pt_j = rowmat(pt)
        px_j = rowmat(px)
        py_j = rowmat(py)
        pz_j = rowmat(pz)
        e_j = rowmat(en)
        keep_j = rowmat(keep)

        phi_i = colmat(phi_t)
        rap_i = colmat(rap_t)
        pt_i = colmat(pt_t)
        px_i = colmat(px_t)
        py_i = colmat(py_t)
        pz_i = colmat(pz_t)
        e_i = colmat(en_t)
        keep_i = colmat(keep_t)

        dphi = jnp.mod(phi_i - phi_j + jnp.pi, 2.0 * jnp.pi) - jnp.pi
        drap = rap_i - rap_j
        dr = jnp.sqrt(drap * drap + dphi * dphi)
        dr = jnp.log(1.0 + jnp.clip(dr, _EPS, None))
        minpt = jnp.minimum(pt_i, pt_j)
        kt = jnp.log(1.0 + jnp.clip(minpt * dr, _EPS, None))
        z = jnp.log(1.0 + jnp.clip(minpt / (pt_i + pt_j + _EPS), _EPS, None))
        se = e_i + e_j
        spx = px_i + px_j
        spy = py_i + py_j
        spz = pz_i + pz_j
        m2 = jnp.log(1.0 + jnp.clip(
            se * se - spx * spx - spy * spy - spz * spz, _EPS, None))

        scale = offdiag * keep_i * keep_j
        dr_ref[r] = (dr * scale).astype(jnp.bfloat16)
        kt_ref[r] = (kt * scale).astype(jnp.bfloat16)
        z_ref[r] = (z * scale).astype(jnp.bfloat16)
        m2_ref[r] = (m2 * scale).astype(jnp.bfloat16)


def kernel(pt, eta, phi, energy, mask):
    b, n = pt.shape
    maskf = mask.astype(jnp.float32)
    bspec_in = pl.BlockSpec((_BB, n), lambda g: (g, 0))
    bspec_out = pl.BlockSpec((_BB, n, n), lambda g: (g, 0, 0))
    shp = jax.ShapeDtypeStruct((b, n, n), jnp.bfloat16)
    dr, kt, z, m2 = pl.pallas_call(
        _feat_kernel,
        grid=(b // _BB,),
        in_specs=[bspec_in] * 5,
        out_specs=[bspec_out] * 4,
        out_shape=[shp] * 4,
    )(pt, eta, phi, energy, maskf)
    features = jnp.stack([dr, kt, z, m2], axis=-1).astype(jnp.float32)
    pair_mask = mask[:, _II] | mask[:, _JJ]
    return features, pair_mask
